# Initial kernel scaffold; baseline (speedup 1.0000x reference)
import jax
import jax.numpy as jnp
from jax.experimental import pallas as pl


def kernel(x, edge_index, edge_weight, batch_index, W0, b0, W1, b1, W2, b2, Wout, bout):
    raise NotImplementedError("write your pallas kernel here")


# speedup vs baseline: 15.6120x; 15.6120x over previous
import functools

import jax
import jax.numpy as jnp
from jax import lax
from jax.experimental import pallas as pl
from jax.experimental.pallas import tpu as pltpu
from jax.experimental.pallas import tpu_sc as plsc

NC = 2
NS = 16
NW = NC * NS
LANES = 16


def _sc_mesh():
  return plsc.VectorSubcoreMesh(core_axis_name="c", subcore_axis_name="s")


_SPLAT_DNUMS = lax.GatherDimensionNumbers(
    offset_dims=(), collapsed_slice_dims=(0,), start_index_map=(0,))


def _lane_splat(vec, lane):
  idx = jnp.full((LANES, 1), lane, dtype=jnp.int32)
  return lax.gather(vec, idx, _SPLAT_DNUMS, (1,),
                    mode=lax.GatherScatterMode.PROMISE_IN_BOUNDS)


def _make_deg_kernel(NP, NCH, CH):
  npart = NP // NS

  def body(dst_hbm, ew_hbm, deg_hbm, dst_v, ew_v, zb_v, deg_sp):
    c = lax.axis_index("c")
    s = lax.axis_index("s")
    w = c * NS + s
    z16 = jnp.zeros((LANES,), jnp.float32)
    for i in range(zb_v.shape[0] // LANES):
      zb_v[pl.ds(i * LANES, LANES)] = z16
    for k in range(npart // zb_v.shape[0]):
      pltpu.sync_copy(zb_v, deg_sp.at[pl.ds(s * npart + k * zb_v.shape[0],
                                            zb_v.shape[0])])
    plsc.subcore_barrier()
    pltpu.sync_copy(dst_hbm.at[w], dst_v)
    pltpu.sync_copy(ew_hbm.at[w], ew_v)

    def chunk(j, carry):
      pltpu.sync_copy(ew_v.at[j], deg_sp.at[dst_v.at[j]], add=True)
      return carry

    lax.fori_loop(0, NCH, chunk, 0)
    plsc.subcore_barrier()
    pltpu.sync_copy(deg_sp.at[pl.ds(s * npart, npart)],
                    deg_hbm.at[c, pl.ds(s * npart, npart)])

  return pl.kernel(
      body,
      out_type=jax.ShapeDtypeStruct((NC, NP), jnp.float32),
      mesh=_sc_mesh(),
      compiler_params=pltpu.CompilerParams(use_tc_tiling_on_sc=False),
      scratch_types=[
          pltpu.VMEM((NCH, CH), jnp.int32),
          pltpu.VMEM((NCH, CH), jnp.float32),
          pltpu.VMEM((640,), jnp.float32),
          pltpu.VMEM_SHARED((NP,), jnp.float32),
      ],
  )


def _make_agg_kernel(NP, H, NCH, CH):
  npart = NP // NS
  zrows = 80

  def body(hp_hbm, src_hbm, dst_hbm, ew_hbm, s_hbm,
           src_v, dst_v, ew_v, rows_v, zb_v, s_sp, sem):
    c = lax.axis_index("c")
    s = lax.axis_index("s")
    w = c * NS + s
    z16 = jnp.zeros((LANES,), jnp.float32)
    for r in range(zrows):
      for f in range(H // LANES):
        zb_v[r, pl.ds(f * LANES, LANES)] = z16
    for k in range(npart // zrows):
      pltpu.sync_copy(zb_v, s_sp.at[pl.ds(s * npart + k * zrows, zrows)])
    plsc.subcore_barrier()
    pltpu.sync_copy(src_hbm.at[w], src_v)
    pltpu.sync_copy(dst_hbm.at[w], dst_v)
    pltpu.sync_copy(ew_hbm.at[w], ew_v)

    def chunk(j, carry):
      pltpu.async_copy(hp_hbm.at[src_v.at[j]], rows_v, sem).wait()
      for g in range(CH // LANES):
        ew16 = ew_v[j, pl.ds(g * LANES, LANES)]
        for l in range(LANES):
          e = g * LANES + l
          scl = _lane_splat(ew16, l)
          for f in range(H // LANES):
            sl = pl.ds(f * LANES, LANES)
            rows_v[e, sl] = rows_v[e, sl] * scl
      pltpu.sync_copy(rows_v, s_sp.at[dst_v.at[j]], add=True)
      return carry

    lax.fori_loop(0, NCH, chunk, 0)
    plsc.subcore_barrier()
    for k in range(npart // zrows):
      pltpu.sync_copy(s_sp.at[pl.ds(s * npart + k * zrows, zrows)],
                      s_hbm.at[c, pl.ds(s * npart + k * zrows, zrows)])

  return pl.kernel(
      body,
      out_type=jax.ShapeDtypeStruct((NC, NP, H), jnp.float32),
      mesh=_sc_mesh(),
      compiler_params=pltpu.CompilerParams(use_tc_tiling_on_sc=False),
      scratch_types=[
          pltpu.VMEM((NCH, CH), jnp.int32),
          pltpu.VMEM((NCH, CH), jnp.int32),
          pltpu.VMEM((NCH, CH), jnp.float32),
          pltpu.VMEM((CH, H), jnp.float32),
          pltpu.VMEM((zrows, H), jnp.float32),
          pltpu.VMEM_SHARED((NP, H), jnp.float32),
          pltpu.SemaphoreType.DMA,
      ],
  )


def _tc_dis_body(degp_ref, dis_ref):
  deg = degp_ref[0] + degp_ref[1] + 1.0
  dis_ref[...] = lax.rsqrt(deg)


def _tc_first_body(x_ref, w_ref, dis_ref, hp_ref):
  h = jnp.dot(x_ref[...], w_ref[...], preferred_element_type=jnp.float32)
  hp_ref[...] = dis_ref[...] * h


def _tc_mid_body(sp_ref, hp_ref, dis_ref, b_ref, w_ref, out_ref):
  H = w_ref.shape[0]
  dis = dis_ref[...]
  agg = dis * (sp_ref[0, :, :H] + sp_ref[1, :, :H] + hp_ref[:, :H]) + b_ref[...]
  h = jnp.maximum(agg, 0.0)
  out_ref[...] = dis * jnp.dot(h, w_ref[...],
                               preferred_element_type=jnp.float32)


def _make_tc_final_body(G):
  def body(sp_ref, hp_ref, dis_ref, b_ref, brow_ref, bcol_ref,
           wout_ref, bout_ref, out_ref, hidden_ref):
    H = b_ref.shape[1]
    dis = dis_ref[...]
    agg = dis * (sp_ref[0, :, :H] + sp_ref[1, :, :H] + hp_ref[:, :H]) + b_ref[...]
    h3 = jnp.maximum(agg, 0.0)
    NP, H = h3.shape
    gids = lax.broadcasted_iota(jnp.int32, (G, NP), 0)
    p = (gids == brow_ref[...]).astype(jnp.float32)
    gsum = jnp.dot(p, h3, preferred_element_type=jnp.float32)
    cnt = jnp.dot(p, jnp.ones((NP, 1), jnp.float32),
                  preferred_element_type=jnp.float32)
    bcol = bcol_ref[...]

    grow = lax.broadcasted_iota(jnp.int32, (G, 1), 0)

    def gstep(g, acc):
      m = jnp.max(jnp.where(bcol == g, h3, -jnp.inf), axis=0)
      return jnp.where(grow == g, m[None, :], acc)

    gmax = lax.fori_loop(0, G, gstep, jnp.zeros((G, H), jnp.float32))
    gmean = gsum / jnp.maximum(cnt, 1.0)
    hidden = jnp.concatenate([gmax, gmean], axis=1)
    logit = jnp.dot(hidden, wout_ref[...],
                    preferred_element_type=jnp.float32) + bout_ref[...]
    out_ref[...] = jax.nn.sigmoid(logit)
    hidden_ref[...] = hidden
  return body


def _tc_call(body, out_shape):
  return pl.pallas_call(body, out_shape=out_shape)


@jax.jit
def kernel(x, edge_index, edge_weight, batch_index,
           W0, b0, W1, b1, W2, b2, Wout, bout):
  N, F = x.shape
  E = edge_index.shape[1]
  H = W0.shape[1]
  G = 64
  NP = ((N + 1023) // 1024) * 1024
  EPW = E // NW
  CH = 80
  NCH = EPW // CH
  assert E % NW == 0 and EPW % CH == 0 and NP % (NS * 80) == 0

  f32 = jnp.float32
  src_r = edge_index[0].reshape(NW, NCH, CH)
  dst_r = edge_index[1].reshape(NW, NCH, CH)
  ew_r = edge_weight.reshape(NW, NCH, CH)
  x_p = jnp.zeros((NP, F), f32).at[:N].set(x)
  batch_p = jnp.full((NP,), G, jnp.int32).at[:N].set(batch_index)
  brow = batch_p.reshape(1, NP)
  bcol = batch_p.reshape(NP, 1)

  deg_kernel = _make_deg_kernel(NP, NCH, CH)
  agg_kernel = _make_agg_kernel(NP, H, NCH, CH)

  degp = deg_kernel(dst_r, ew_r)
  dis = _tc_call(_tc_dis_body,
                 jax.ShapeDtypeStruct((NP // 128, 128), f32))(
                     degp.reshape(NC, NP // 128, 128))
  dis_col = dis.reshape(NP, 1)

  hp0 = _tc_call(_tc_first_body, jax.ShapeDtypeStruct((NP, H), f32))(
      x_p, W0, dis_col)

  wstack = jnp.stack([W1, W2, W2])
  bstack = jnp.stack([b0.reshape(1, H), b1.reshape(1, H), b2.reshape(1, H)])

  def layer(l, carry):
    hp, _, _ = carry
    s = agg_kernel(hp, src_r, dst_r, ew_r)
    hp_next = _tc_call(_tc_mid_body, jax.ShapeDtypeStruct((NP, H), f32))(
        s, hp, dis_col, bstack[l], wstack[l])
    return (hp_next, s, hp)

  s_dummy = jnp.zeros((NC, NP, H), f32)
  _, s2, hp2 = lax.fori_loop(0, 3, layer, (hp0, s_dummy, hp0))

  out, hidden = _tc_call(
      _make_tc_final_body(G),
      (jax.ShapeDtypeStruct((G, 1), f32),
       jax.ShapeDtypeStruct((G, 2 * H), f32)))(
           s2, hp2, dis_col, b2.reshape(1, H), brow, bcol, Wout,
           bout.reshape(1, 1))
  return (out, hidden)


# speedup vs baseline: 23.3902x; 1.4982x over previous
import functools

import jax
import jax.numpy as jnp
from jax import lax
from jax.experimental import pallas as pl
from jax.experimental.pallas import tpu as pltpu
from jax.experimental.pallas import tpu_sc as plsc

NC = 2
NS = 16
NW = NC * NS
LANES = 16


def _sc_mesh():
  return plsc.VectorSubcoreMesh(core_axis_name="c", subcore_axis_name="s")


_SPLAT_DNUMS = lax.GatherDimensionNumbers(
    offset_dims=(), collapsed_slice_dims=(0,), start_index_map=(0,))


def _lane_splat(vec, lane):
  idx = jnp.full((LANES, 1), lane, dtype=jnp.int32)
  return lax.gather(vec, idx, _SPLAT_DNUMS, (1,),
                    mode=lax.GatherScatterMode.PROMISE_IN_BOUNDS)


def _make_deg_kernel(NP, NCH, CH):
  npart = NP // NS

  def body(dst_hbm, ew_hbm, deg_hbm, dst_v, ew_v, zb_v, deg_sp):
    c = lax.axis_index("c")
    s = lax.axis_index("s")
    w = c * NS + s
    z16 = jnp.zeros((LANES,), jnp.float32)
    for i in range(zb_v.shape[0] // LANES):
      zb_v[pl.ds(i * LANES, LANES)] = z16
    for k in range(npart // zb_v.shape[0]):
      pltpu.sync_copy(zb_v, deg_sp.at[pl.ds(s * npart + k * zb_v.shape[0],
                                            zb_v.shape[0])])
    plsc.subcore_barrier()
    pltpu.sync_copy(dst_hbm.at[w], dst_v)
    pltpu.sync_copy(ew_hbm.at[w], ew_v)

    def chunk(j, carry):
      pltpu.sync_copy(ew_v.at[j], deg_sp.at[dst_v.at[j]], add=True)
      return carry

    lax.fori_loop(0, NCH, chunk, 0)
    plsc.subcore_barrier()
    pltpu.sync_copy(deg_sp.at[pl.ds(s * npart, npart)],
                    deg_hbm.at[c, pl.ds(s * npart, npart)])

  return pl.kernel(
      body,
      out_type=jax.ShapeDtypeStruct((NC, NP), jnp.float32),
      mesh=_sc_mesh(),
      compiler_params=pltpu.CompilerParams(use_tc_tiling_on_sc=False),
      scratch_types=[
          pltpu.VMEM((NCH, CH), jnp.int32),
          pltpu.VMEM((NCH, CH), jnp.float32),
          pltpu.VMEM((640,), jnp.float32),
          pltpu.VMEM_SHARED((NP,), jnp.float32),
      ],
  )


def _make_agg_kernel(NP, H, NCH, CH):
  npart = NP // NS
  zrows = 80

  NB = 5
  steps = NCH // NB

  def body(hp_hbm, src_hbm, dst_hbm, ew_hbm, s_hbm,
           src_v, dst_v, ew_v, rows_v, sc_v, zb_v, s_sp, gsem, ssem):
    c = lax.axis_index("c")
    s = lax.axis_index("s")
    w = c * NS + s
    z16 = jnp.zeros((LANES,), jnp.float32)
    for r in range(zrows):
      for f in range(H // LANES):
        zb_v[r, pl.ds(f * LANES, LANES)] = z16
    for k in range(npart // zrows):
      pltpu.sync_copy(zb_v, s_sp.at[pl.ds(s * npart + k * zrows, zrows)])
    plsc.subcore_barrier()
    pltpu.sync_copy(src_hbm.at[w], src_v)
    pltpu.sync_copy(dst_hbm.at[w], dst_v)
    pltpu.sync_copy(ew_hbm.at[w], ew_v)

    for b in range(NB):
      pltpu.async_copy(hp_hbm.at[src_v.at[b]], rows_v.at[b], gsem.at[b])

    def scale(j, b):
      for g in range(CH // LANES):
        ew16 = ew_v[j, pl.ds(g * LANES, LANES)]
        for l in range(LANES):
          e = g * LANES + l
          scl = _lane_splat(ew16, l)
          for f in range(H // LANES):
            sl = pl.ds(f * LANES, LANES)
            sc_v[b, e, sl] = rows_v[b, e, sl] * scl

    def step(t, carry):
      for b in range(NB):
        j = t * NB + b
        pltpu.make_async_copy(hp_hbm.at[src_v.at[j]], rows_v.at[b],
                              gsem.at[b]).wait()

        @pl.when(t > 0)
        def _wait_prev_scatter():
          pltpu.make_async_copy(sc_v.at[b], s_sp.at[dst_v.at[j]],
                                ssem.at[b]).wait()

        scale(j, b)

        @pl.when(t + 1 < steps)
        def _next_gather():
          pltpu.async_copy(hp_hbm.at[src_v.at[j + NB]], rows_v.at[b],
                           gsem.at[b])

        pltpu.async_copy(sc_v.at[b], s_sp.at[dst_v.at[j]], ssem.at[b],
                         add=True)
      return carry

    lax.fori_loop(0, steps, step, 0)
    for b in range(NB):
      pltpu.make_async_copy(sc_v.at[b], s_sp.at[dst_v.at[b]],
                            ssem.at[b]).wait()
    plsc.subcore_barrier()
    for k in range(npart // zrows):
      pltpu.sync_copy(s_sp.at[pl.ds(s * npart + k * zrows, zrows)],
                      s_hbm.at[c, pl.ds(s * npart + k * zrows, zrows)])

  return pl.kernel(
      body,
      out_type=jax.ShapeDtypeStruct((NC, NP, H), jnp.float32),
      mesh=_sc_mesh(),
      compiler_params=pltpu.CompilerParams(use_tc_tiling_on_sc=False),
      scratch_types=[
          pltpu.VMEM((NCH, CH), jnp.int32),
          pltpu.VMEM((NCH, CH), jnp.int32),
          pltpu.VMEM((NCH, CH), jnp.float32),
          pltpu.VMEM((NB, CH, H), jnp.float32),
          pltpu.VMEM((NB, CH, H), jnp.float32),
          pltpu.VMEM((zrows, H), jnp.float32),
          pltpu.VMEM_SHARED((NP, H), jnp.float32),
          pltpu.SemaphoreType.DMA((NB,)),
          pltpu.SemaphoreType.DMA((NB,)),
      ],
  )


def _tc_dis_body(degp_ref, dis_ref):
  deg = degp_ref[0] + degp_ref[1] + 1.0
  dis_ref[...] = lax.rsqrt(deg)


def _tc_first_body(x_ref, w_ref, dis_ref, hp_ref):
  h = jnp.dot(x_ref[...], w_ref[...], preferred_element_type=jnp.float32)
  hp_ref[...] = dis_ref[...] * h


def _tc_mid_body(sp_ref, hp_ref, dis_ref, b_ref, w_ref, out_ref):
  H = w_ref.shape[0]
  dis = dis_ref[...]
  agg = dis * (sp_ref[0, :, :H] + sp_ref[1, :, :H] + hp_ref[:, :H]) + b_ref[...]
  h = jnp.maximum(agg, 0.0)
  out_ref[...] = dis * jnp.dot(h, w_ref[...],
                               preferred_element_type=jnp.float32)


def _make_tc_final_body(G):
  def body(sp_ref, hp_ref, dis_ref, b_ref, brow_ref, bcol_ref,
           wout_ref, bout_ref, out_ref, hidden_ref):
    H = b_ref.shape[1]
    dis = dis_ref[...]
    agg = dis * (sp_ref[0, :, :H] + sp_ref[1, :, :H] + hp_ref[:, :H]) + b_ref[...]
    h3 = jnp.maximum(agg, 0.0)
    NP, H = h3.shape
    gids = lax.broadcasted_iota(jnp.int32, (G, NP), 0)
    p = (gids == brow_ref[...]).astype(jnp.float32)
    gsum = jnp.dot(p, h3, preferred_element_type=jnp.float32)
    cnt = jnp.dot(p, jnp.ones((NP, 1), jnp.float32),
                  preferred_element_type=jnp.float32)
    bcol = bcol_ref[...]

    grow = lax.broadcasted_iota(jnp.int32, (G, 1), 0)

    def gstep(g, acc):
      m = jnp.max(jnp.where(bcol == g, h3, -jnp.inf), axis=0)
      return jnp.where(grow == g, m[None, :], acc)

    gmax = lax.fori_loop(0, G, gstep, jnp.zeros((G, H), jnp.float32))
    gmean = gsum / jnp.maximum(cnt, 1.0)
    hidden = jnp.concatenate([gmax, gmean], axis=1)
    logit = jnp.dot(hidden, wout_ref[...],
                    preferred_element_type=jnp.float32) + bout_ref[...]
    out_ref[...] = jax.nn.sigmoid(logit)
    hidden_ref[...] = hidden
  return body


def _tc_call(body, out_shape):
  return pl.pallas_call(body, out_shape=out_shape)


@jax.jit
def kernel(x, edge_index, edge_weight, batch_index,
           W0, b0, W1, b1, W2, b2, Wout, bout):
  N, F = x.shape
  E = edge_index.shape[1]
  H = W0.shape[1]
  G = 64
  NP = ((N + 1023) // 1024) * 1024
  EPW = E // NW
  CH = 80
  NCH = EPW // CH
  assert E % NW == 0 and EPW % CH == 0 and NP % (NS * 80) == 0

  f32 = jnp.float32
  src_r = edge_index[0].reshape(NW, NCH, CH)
  dst_r = edge_index[1].reshape(NW, NCH, CH)
  ew_r = edge_weight.reshape(NW, NCH, CH)
  x_p = jnp.zeros((NP, F), f32).at[:N].set(x)
  batch_p = jnp.full((NP,), G, jnp.int32).at[:N].set(batch_index)
  brow = batch_p.reshape(1, NP)
  bcol = batch_p.reshape(NP, 1)

  deg_kernel = _make_deg_kernel(NP, NCH, CH)
  agg_kernel = _make_agg_kernel(NP, H, NCH, CH)

  degp = deg_kernel(dst_r, ew_r)
  dis = _tc_call(_tc_dis_body,
                 jax.ShapeDtypeStruct((NP // 128, 128), f32))(
                     degp.reshape(NC, NP // 128, 128))
  dis_col = dis.reshape(NP, 1)

  hp0 = _tc_call(_tc_first_body, jax.ShapeDtypeStruct((NP, H), f32))(
      x_p, W0, dis_col)

  wstack = jnp.stack([W1, W2, W2])
  bstack = jnp.stack([b0.reshape(1, H), b1.reshape(1, H), b2.reshape(1, H)])

  def layer(l, carry):
    hp, _, _ = carry
    s = agg_kernel(hp, src_r, dst_r, ew_r)
    hp_next = _tc_call(_tc_mid_body, jax.ShapeDtypeStruct((NP, H), f32))(
        s, hp, dis_col, bstack[l], wstack[l])
    return (hp_next, s, hp)

  s_dummy = jnp.zeros((NC, NP, H), f32)
  _, s2, hp2 = lax.fori_loop(0, 3, layer, (hp0, s_dummy, hp0))

  out, hidden = _tc_call(
      _make_tc_final_body(G),
      (jax.ShapeDtypeStruct((G, 1), f32),
       jax.ShapeDtypeStruct((G, 2 * H), f32)))(
           s2, hp2, dis_col, b2.reshape(1, H), brow, bcol, Wout,
           bout.reshape(1, 1))
  return (out, hidden)


# speedup vs baseline: 24.3965x; 1.0430x over previous
import functools

import jax
import jax.numpy as jnp
from jax import lax
from jax.experimental import pallas as pl
from jax.experimental.pallas import tpu as pltpu
from jax.experimental.pallas import tpu_sc as plsc

NC = 2
NS = 16
NW = NC * NS
LANES = 16


def _sc_mesh():
  return plsc.VectorSubcoreMesh(core_axis_name="c", subcore_axis_name="s")


_SPLAT_DNUMS = lax.GatherDimensionNumbers(
    offset_dims=(), collapsed_slice_dims=(0,), start_index_map=(0,))


def _lane_splat(vec, lane):
  idx = jnp.full((LANES, 1), lane, dtype=jnp.int32)
  return lax.gather(vec, idx, _SPLAT_DNUMS, (1,),
                    mode=lax.GatherScatterMode.PROMISE_IN_BOUNDS)


def _make_deg_kernel(NP, NCH, CH):
  npart = NP // NS

  def body(dst_hbm, ew_hbm, deg_hbm, dst_v, ew_v, zb_v, deg_sp):
    c = lax.axis_index("c")
    s = lax.axis_index("s")
    w = c * NS + s
    z16 = jnp.zeros((LANES,), jnp.float32)
    for i in range(zb_v.shape[0] // LANES):
      zb_v[pl.ds(i * LANES, LANES)] = z16
    for k in range(npart // zb_v.shape[0]):
      pltpu.sync_copy(zb_v, deg_sp.at[pl.ds(s * npart + k * zb_v.shape[0],
                                            zb_v.shape[0])])
    plsc.subcore_barrier()
    pltpu.sync_copy(dst_hbm.at[w], dst_v)
    pltpu.sync_copy(ew_hbm.at[w], ew_v)

    def chunk(j, carry):
      pltpu.sync_copy(ew_v.at[j], deg_sp.at[dst_v.at[j]], add=True)
      return carry

    lax.fori_loop(0, NCH, chunk, 0)
    plsc.subcore_barrier()
    pltpu.sync_copy(deg_sp.at[pl.ds(s * npart, npart)],
                    deg_hbm.at[c, pl.ds(s * npart, npart)])

  return pl.kernel(
      body,
      out_type=jax.ShapeDtypeStruct((NC, NP), jnp.float32),
      mesh=_sc_mesh(),
      compiler_params=pltpu.CompilerParams(use_tc_tiling_on_sc=False),
      scratch_types=[
          pltpu.VMEM((NCH, CH), jnp.int32),
          pltpu.VMEM((NCH, CH), jnp.float32),
          pltpu.VMEM((640,), jnp.float32),
          pltpu.VMEM_SHARED((NP,), jnp.float32),
      ],
  )


def _make_agg_kernel(NP, H, NCH, CH):
  npart = NP // NS
  zrows = 80

  NB = 5
  steps = NCH // NB

  def body(hp_hbm, src_hbm, dst_hbm, ew_hbm, s_hbm,
           src_v, dst_v, ew_v, rows_v, sc_v, zb_v, s_sp, gsem, ssem):
    c = lax.axis_index("c")
    s = lax.axis_index("s")
    w = c * NS + s
    z16 = jnp.zeros((LANES,), jnp.float32)
    for r in range(zrows):
      for f in range(H // LANES):
        zb_v[r, pl.ds(f * LANES, LANES)] = z16
    for k in range(npart // zrows):
      pltpu.sync_copy(zb_v, s_sp.at[pl.ds(s * npart + k * zrows, zrows)])
    plsc.subcore_barrier()
    pltpu.sync_copy(src_hbm.at[w], src_v)
    pltpu.sync_copy(dst_hbm.at[w], dst_v)
    pltpu.sync_copy(ew_hbm.at[w], ew_v)

    for b in range(NB):
      pltpu.async_copy(hp_hbm.at[src_v.at[b]], rows_v.at[b], gsem.at[b])

    def scale(j, b):
      for g in range(CH // LANES):
        ew16 = ew_v[j, pl.ds(g * LANES, LANES)]
        for l in range(LANES):
          e = g * LANES + l
          scl = _lane_splat(ew16, l)
          for f in range(H // LANES):
            sl = pl.ds(f * LANES, LANES)
            sc_v[b, e, sl] = rows_v[b, e, sl] * scl

    def step(t, carry):
      for b in range(NB):
        j = t * NB + b
        pltpu.make_async_copy(hp_hbm.at[src_v.at[j]], rows_v.at[b],
                              gsem.at[b]).wait()

        @pl.when(t > 0)
        def _wait_prev_scatter():
          pltpu.make_async_copy(sc_v.at[b], s_sp.at[dst_v.at[j]],
                                ssem.at[b]).wait()

        scale(j, b)

        @pl.when(t + 1 < steps)
        def _next_gather():
          pltpu.async_copy(hp_hbm.at[src_v.at[j + NB]], rows_v.at[b],
                           gsem.at[b])

        pltpu.async_copy(sc_v.at[b], s_sp.at[dst_v.at[j]], ssem.at[b],
                         add=True)
      return carry

    lax.fori_loop(0, steps, step, 0)
    for b in range(NB):
      pltpu.make_async_copy(sc_v.at[b], s_sp.at[dst_v.at[b]],
                            ssem.at[b]).wait()
    plsc.subcore_barrier()
    for k in range(npart // zrows):
      pltpu.sync_copy(s_sp.at[pl.ds(s * npart + k * zrows, zrows)],
                      s_hbm.at[c, pl.ds(s * npart + k * zrows, zrows)])

  return pl.kernel(
      body,
      out_type=jax.ShapeDtypeStruct((NC, NP, H), jnp.float32),
      mesh=_sc_mesh(),
      compiler_params=pltpu.CompilerParams(use_tc_tiling_on_sc=False),
      scratch_types=[
          pltpu.VMEM((NCH, CH), jnp.int32),
          pltpu.VMEM((NCH, CH), jnp.int32),
          pltpu.VMEM((NCH, CH), jnp.float32),
          pltpu.VMEM((NB, CH, H), jnp.float32),
          pltpu.VMEM((NB, CH, H), jnp.float32),
          pltpu.VMEM((zrows, H), jnp.float32),
          pltpu.VMEM_SHARED((NP, H), jnp.float32),
          pltpu.SemaphoreType.DMA((NB,)),
          pltpu.SemaphoreType.DMA((NB,)),
      ],
  )


def _tc_dis_body(degp_ref, dis_ref):
  deg = degp_ref[0] + degp_ref[1] + 1.0
  dis_ref[...] = lax.rsqrt(deg)


def _tc_first_body(x_ref, w_ref, dis_ref, hp_ref):
  h = jnp.dot(x_ref[...], w_ref[...], preferred_element_type=jnp.float32)
  hp_ref[...] = dis_ref[...] * h


def _tc_mid_body(sp_ref, hp_ref, dis_ref, b_ref, w_ref, out_ref):
  H = w_ref.shape[0]
  dis = dis_ref[...]
  agg = dis * (sp_ref[0, :, :H] + sp_ref[1, :, :H] + hp_ref[:, :H]) + b_ref[...]
  h = jnp.maximum(agg, 0.0)
  out_ref[...] = dis * jnp.dot(h, w_ref[...],
                               preferred_element_type=jnp.float32)


def _make_tc_final_body(G):
  def body(sp_ref, hp_ref, dis_ref, b_ref, brow_ref, bcol_ref,
           wout_ref, bout_ref, out_ref, hidden_ref):
    H = b_ref.shape[1]
    dis = dis_ref[...]
    agg = dis * (sp_ref[0, :, :H] + sp_ref[1, :, :H] + hp_ref[:, :H]) + b_ref[...]
    h3 = jnp.maximum(agg, 0.0)
    NP, H = h3.shape
    gids = lax.broadcasted_iota(jnp.int32, (G, NP), 0)
    p = (gids == brow_ref[...]).astype(jnp.float32)
    gsum = jnp.dot(p, h3, preferred_element_type=jnp.float32)
    cnt = jnp.dot(p, jnp.ones((NP, 1), jnp.float32),
                  preferred_element_type=jnp.float32)
    bcol = bcol_ref[...]

    grow = lax.broadcasted_iota(jnp.int32, (G, 1), 0)

    def gstep(g, acc):
      m = jnp.max(jnp.where(bcol == g, h3, -jnp.inf), axis=0)
      return jnp.where(grow == g, m[None, :], acc)

    gmax = lax.fori_loop(0, G, gstep, jnp.zeros((G, H), jnp.float32))
    gmean = gsum / jnp.maximum(cnt, 1.0)
    hidden = jnp.concatenate([gmax, gmean], axis=1)
    logit = jnp.dot(hidden, wout_ref[...],
                    preferred_element_type=jnp.float32) + bout_ref[...]
    out_ref[...] = jax.nn.sigmoid(logit)
    hidden_ref[...] = hidden
  return body


def _tc_call(body, out_shape):
  return pl.pallas_call(body, out_shape=out_shape)


@jax.jit
def kernel(x, edge_index, edge_weight, batch_index,
           W0, b0, W1, b1, W2, b2, Wout, bout):
  N, F = x.shape
  E = edge_index.shape[1]
  H = W0.shape[1]
  G = 64
  NP = ((N + 1023) // 1024) * 1024
  EPW = E // NW
  CH = 80
  NCH = EPW // CH
  assert E % NW == 0 and EPW % CH == 0 and NP % (NS * 80) == 0

  f32 = jnp.float32
  src_r = edge_index[0].reshape(NW, NCH, CH)
  dst_r = edge_index[1].reshape(NW, NCH, CH)
  ew_r = edge_weight.reshape(NW, NCH, CH)
  x_p = jnp.zeros((NP, F), f32).at[:N].set(x)
  batch_p = jnp.full((NP,), G, jnp.int32).at[:N].set(batch_index)
  brow = batch_p.reshape(1, NP)
  bcol = batch_p.reshape(NP, 1)

  deg_kernel = _make_deg_kernel(NP, NCH, CH)
  agg_kernel = _make_agg_kernel(NP, H, NCH, CH)

  degp = deg_kernel(dst_r, ew_r)
  dis = _tc_call(_tc_dis_body,
                 jax.ShapeDtypeStruct((NP // 128, 128), f32))(
                     degp.reshape(NC, NP // 128, 128))
  dis_col = dis.reshape(NP, 1)

  hp0 = _tc_call(_tc_first_body, jax.ShapeDtypeStruct((NP, H), f32))(
      x_p, W0, dis_col)

  s0 = agg_kernel(hp0, src_r, dst_r, ew_r)
  hp1 = _tc_call(_tc_mid_body, jax.ShapeDtypeStruct((NP, H), f32))(
      s0, hp0, dis_col, b0.reshape(1, H), W1)
  s1 = agg_kernel(hp1, src_r, dst_r, ew_r)
  hp2 = _tc_call(_tc_mid_body, jax.ShapeDtypeStruct((NP, H), f32))(
      s1, hp1, dis_col, b1.reshape(1, H), W2)
  s2 = agg_kernel(hp2, src_r, dst_r, ew_r)

  out, hidden = _tc_call(
      _make_tc_final_body(G),
      (jax.ShapeDtypeStruct((G, 1), f32),
       jax.ShapeDtypeStruct((G, 2 * H), f32)))(
           s2, hp2, dis_col, b2.reshape(1, H), brow, bcol, Wout,
           bout.reshape(1, 1))
  return (out, hidden)


# speedup vs baseline: 29.6855x; 1.2168x over previous
import functools

import jax
import jax.numpy as jnp
from jax import lax
from jax.experimental import pallas as pl
from jax.experimental.pallas import tpu as pltpu
from jax.experimental.pallas import tpu_sc as plsc

NC = 2
NS = 16
NW = NC * NS
LANES = 16


def _sc_mesh():
  return plsc.VectorSubcoreMesh(core_axis_name="c", subcore_axis_name="s")


_SPLAT_DNUMS = lax.GatherDimensionNumbers(
    offset_dims=(), collapsed_slice_dims=(0,), start_index_map=(0,))


def _lane_splat(vec, lane):
  idx = jnp.full((LANES, 1), lane, dtype=jnp.int32)
  return lax.gather(vec, idx, _SPLAT_DNUMS, (1,),
                    mode=lax.GatherScatterMode.PROMISE_IN_BOUNDS)


def _make_deg_kernel(NP, NCH, CH):
  npart = NP // NS

  def body(dst_hbm, ew_hbm, deg_hbm, dst_v, ew_v, zb_v, deg_sp):
    c = lax.axis_index("c")
    s = lax.axis_index("s")
    w = c * NS + s
    z16 = jnp.zeros((LANES,), jnp.float32)
    for i in range(zb_v.shape[0] // LANES):
      zb_v[pl.ds(i * LANES, LANES)] = z16
    for k in range(npart // zb_v.shape[0]):
      pltpu.sync_copy(zb_v, deg_sp.at[pl.ds(s * npart + k * zb_v.shape[0],
                                            zb_v.shape[0])])
    plsc.subcore_barrier()
    pltpu.sync_copy(dst_hbm.at[w], dst_v)
    pltpu.sync_copy(ew_hbm.at[w], ew_v)

    def chunk(j, carry):
      pltpu.sync_copy(ew_v.at[j], deg_sp.at[dst_v.at[j]], add=True)
      return carry

    lax.fori_loop(0, NCH, chunk, 0)
    plsc.subcore_barrier()
    pltpu.sync_copy(deg_sp.at[pl.ds(s * npart, npart)],
                    deg_hbm.at[c, pl.ds(s * npart, npart)])

  return pl.kernel(
      body,
      out_type=jax.ShapeDtypeStruct((NC, NP), jnp.float32),
      mesh=_sc_mesh(),
      compiler_params=pltpu.CompilerParams(use_tc_tiling_on_sc=False),
      scratch_types=[
          pltpu.VMEM((NCH, CH), jnp.int32),
          pltpu.VMEM((NCH, CH), jnp.float32),
          pltpu.VMEM((640,), jnp.float32),
          pltpu.VMEM_SHARED((NP,), jnp.float32),
      ],
  )


def _make_agg_kernel(NP, H, NCH, CH):
  npart = NP // NS
  zrows = 80

  NB = 5
  steps = NCH // NB

  def body(hp_hbm, src_hbm, dst_hbm, ew_hbm, s_hbm,
           src_v, dst_v, ew_v, rows_v, sc_v, zb_v, s_sp, gsem, ssem):
    c = lax.axis_index("c")
    s = lax.axis_index("s")
    w = c * NS + s
    z16 = jnp.zeros((LANES,), jnp.float32)
    for r in range(zrows):
      for f in range(H // LANES):
        zb_v[r, pl.ds(f * LANES, LANES)] = z16
    for k in range(npart // zrows):
      pltpu.sync_copy(zb_v, s_sp.at[pl.ds(s * npart + k * zrows, zrows)])
    plsc.subcore_barrier()
    pltpu.sync_copy(src_hbm.at[w], src_v)
    pltpu.sync_copy(dst_hbm.at[w], dst_v)
    pltpu.sync_copy(ew_hbm.at[w], ew_v)

    for b in range(NB):
      pltpu.async_copy(hp_hbm.at[src_v.at[b]], rows_v.at[b], gsem.at[b])

    def scale(j, b):
      for g in range(CH // LANES):
        ew16 = ew_v[j, pl.ds(g * LANES, LANES)]
        for l in range(LANES):
          e = g * LANES + l
          scl = _lane_splat(ew16, l)
          for f in range(H // LANES):
            sl = pl.ds(f * LANES, LANES)
            sc_v[b, e, sl] = rows_v[b, e, sl] * scl

    def step(t, carry):
      for b in range(NB):
        j = t * NB + b
        pltpu.make_async_copy(hp_hbm.at[src_v.at[j]], rows_v.at[b],
                              gsem.at[b]).wait()

        @pl.when(t > 0)
        def _wait_prev_scatter():
          pltpu.make_async_copy(sc_v.at[b], s_sp.at[dst_v.at[j]],
                                ssem.at[b]).wait()

        scale(j, b)

        @pl.when(t + 1 < steps)
        def _next_gather():
          pltpu.async_copy(hp_hbm.at[src_v.at[j + NB]], rows_v.at[b],
                           gsem.at[b])

        pltpu.async_copy(sc_v.at[b], s_sp.at[dst_v.at[j]], ssem.at[b],
                         add=True)
      return carry

    lax.fori_loop(0, steps, step, 0)
    for b in range(NB):
      pltpu.make_async_copy(sc_v.at[b], s_sp.at[dst_v.at[b]],
                            ssem.at[b]).wait()
    plsc.subcore_barrier()
    for k in range(npart // zrows):
      pltpu.sync_copy(s_sp.at[pl.ds(s * npart + k * zrows, zrows)],
                      s_hbm.at[c, pl.ds(s * npart + k * zrows, zrows)])

  return pl.kernel(
      body,
      out_type=jax.ShapeDtypeStruct((NC, NP, H), jnp.float32),
      mesh=_sc_mesh(),
      compiler_params=pltpu.CompilerParams(use_tc_tiling_on_sc=False),
      scratch_types=[
          pltpu.VMEM((NCH, CH), jnp.int32),
          pltpu.VMEM((NCH, CH), jnp.int32),
          pltpu.VMEM((NCH, CH), jnp.float32),
          pltpu.VMEM((NB, CH, H), jnp.float32),
          pltpu.VMEM((NB, CH, H), jnp.float32),
          pltpu.VMEM((zrows, H), jnp.float32),
          pltpu.VMEM_SHARED((NP, H), jnp.float32),
          pltpu.SemaphoreType.DMA((NB,)),
          pltpu.SemaphoreType.DMA((NB,)),
      ],
  )


def _tc_dis_body(degp_ref, dis_ref):
  deg = degp_ref[0] + degp_ref[1] + 1.0
  dis_ref[...] = lax.rsqrt(deg)


def _tc_first_body(x_ref, w_ref, dis_ref, hp_ref):
  h = jnp.dot(x_ref[...], w_ref[...], preferred_element_type=jnp.float32)
  hp_ref[...] = dis_ref[...] * h


def _tc_mid_body(sp_ref, hp_ref, dis_ref, b_ref, w_ref, out_ref):
  H = w_ref.shape[0]
  dis = dis_ref[...]
  agg = dis * (sp_ref[0, :, :H] + sp_ref[1, :, :H] + hp_ref[:, :H]) + b_ref[...]
  h = jnp.maximum(agg, 0.0)
  out_ref[...] = dis * jnp.dot(h, w_ref[...],
                               preferred_element_type=jnp.float32)


def _make_tc_final_body(G):
  def body(sp_ref, hp_ref, dis_ref, b_ref, brow_ref, bcol_ref,
           wout_ref, bout_ref, out_ref, hidden_ref):
    H = b_ref.shape[1]
    dis = dis_ref[...]
    agg = dis * (sp_ref[0, :, :H] + sp_ref[1, :, :H] + hp_ref[:, :H]) + b_ref[...]
    h3 = jnp.maximum(agg, 0.0)
    NP, H = h3.shape
    gids = lax.broadcasted_iota(jnp.int32, (G, NP), 0)
    p = (gids == brow_ref[...]).astype(jnp.float32)
    gsum = jnp.dot(p, h3, preferred_element_type=jnp.float32)
    cnt = jnp.dot(p, jnp.ones((NP, 1), jnp.float32),
                  preferred_element_type=jnp.float32)
    bcol = bcol_ref[...]

    grow = lax.broadcasted_iota(jnp.int32, (G, 1), 0)

    CHR = 256
    acc = jnp.full((G, H), -jnp.inf, jnp.float32)
    for i in range(NP // CHR):
      hc = h3[i * CHR:(i + 1) * CHR]
      bc = bcol[i * CHR:(i + 1) * CHR]
      blo = bc[0, 0]
      bhi = bc[CHR - 1, 0]

      def cbody(g, a, hc=hc, bc=bc):
        m = jnp.max(jnp.where(bc == g, hc, -jnp.inf), axis=0)
        return jnp.maximum(a, jnp.where(grow == g, m[None, :], -jnp.inf))

      acc = lax.fori_loop(blo, bhi + 1, cbody, acc)
    gmax = acc
    gmean = gsum / jnp.maximum(cnt, 1.0)
    hidden = jnp.concatenate([gmax, gmean], axis=1)
    logit = jnp.dot(hidden, wout_ref[...],
                    preferred_element_type=jnp.float32) + bout_ref[...]
    out_ref[...] = jax.nn.sigmoid(logit)
    hidden_ref[...] = hidden
  return body


def _tc_call(body, out_shape):
  return pl.pallas_call(body, out_shape=out_shape)


@jax.jit
def kernel(x, edge_index, edge_weight, batch_index,
           W0, b0, W1, b1, W2, b2, Wout, bout):
  N, F = x.shape
  E = edge_index.shape[1]
  H = W0.shape[1]
  G = 64
  NP = ((N + 1023) // 1024) * 1024
  EPW = E // NW
  CH = 80
  NCH = EPW // CH
  assert E % NW == 0 and EPW % CH == 0 and NP % (NS * 80) == 0

  f32 = jnp.float32
  src_r = edge_index[0].reshape(NW, NCH, CH)
  dst_r = edge_index[1].reshape(NW, NCH, CH)
  ew_r = edge_weight.reshape(NW, NCH, CH)
  x_p = jnp.zeros((NP, F), f32).at[:N].set(x)
  batch_p = jnp.full((NP,), G, jnp.int32).at[:N].set(batch_index)
  brow = batch_p.reshape(1, NP)
  bcol = batch_p.reshape(NP, 1)

  deg_kernel = _make_deg_kernel(NP, NCH, CH)
  agg_kernel = _make_agg_kernel(NP, H, NCH, CH)

  degp = deg_kernel(dst_r, ew_r)
  dis = _tc_call(_tc_dis_body,
                 jax.ShapeDtypeStruct((NP // 128, 128), f32))(
                     degp.reshape(NC, NP // 128, 128))
  dis_col = dis.reshape(NP, 1)

  hp0 = _tc_call(_tc_first_body, jax.ShapeDtypeStruct((NP, H), f32))(
      x_p, W0, dis_col)

  s0 = agg_kernel(hp0, src_r, dst_r, ew_r)
  hp1 = _tc_call(_tc_mid_body, jax.ShapeDtypeStruct((NP, H), f32))(
      s0, hp0, dis_col, b0.reshape(1, H), W1)
  s1 = agg_kernel(hp1, src_r, dst_r, ew_r)
  hp2 = _tc_call(_tc_mid_body, jax.ShapeDtypeStruct((NP, H), f32))(
      s1, hp1, dis_col, b1.reshape(1, H), W2)
  s2 = agg_kernel(hp2, src_r, dst_r, ew_r)

  out, hidden = _tc_call(
      _make_tc_final_body(G),
      (jax.ShapeDtypeStruct((G, 1), f32),
       jax.ShapeDtypeStruct((G, 2 * H), f32)))(
           s2, hp2, dis_col, b2.reshape(1, H), brow, bcol, Wout,
           bout.reshape(1, 1))
  return (out, hidden)


# speedup vs baseline: 30.3153x; 1.0212x over previous
import functools

import jax
import jax.numpy as jnp
from jax import lax
from jax.experimental import pallas as pl
from jax.experimental.pallas import tpu as pltpu
from jax.experimental.pallas import tpu_sc as plsc

NC = 2
NS = 16
NW = NC * NS
LANES = 16


def _sc_mesh():
  return plsc.VectorSubcoreMesh(core_axis_name="c", subcore_axis_name="s")


_SPLAT_DNUMS = lax.GatherDimensionNumbers(
    offset_dims=(), collapsed_slice_dims=(0,), start_index_map=(0,))


def _lane_splat(vec, lane):
  idx = jnp.full((LANES, 1), lane, dtype=jnp.int32)
  return lax.gather(vec, idx, _SPLAT_DNUMS, (1,),
                    mode=lax.GatherScatterMode.PROMISE_IN_BOUNDS)


def _make_deg_kernel(NP, NCH, CH):
  npart = NP // NS

  def body(dst_hbm, ew_hbm, deg_hbm, dst_v, ew_v, zb_v, deg_sp, dsem):
    c = lax.axis_index("c")
    s = lax.axis_index("s")
    w = c * NS + s
    z16 = jnp.zeros((LANES,), jnp.float32)
    for i in range(zb_v.shape[0] // LANES):
      zb_v[pl.ds(i * LANES, LANES)] = z16
    for k in range(npart // zb_v.shape[0]):
      pltpu.sync_copy(zb_v, deg_sp.at[pl.ds(s * npart + k * zb_v.shape[0],
                                            zb_v.shape[0])])
    plsc.subcore_barrier()
    pltpu.sync_copy(dst_hbm.at[w], dst_v)
    pltpu.sync_copy(ew_hbm.at[w], ew_v)

    LAG = 8

    def chunk(j, carry):
      pltpu.async_copy(ew_v.at[j], deg_sp.at[dst_v.at[j]], dsem, add=True)

      @pl.when(j >= LAG)
      def _drain():
        pltpu.make_async_copy(ew_v.at[j - LAG], deg_sp.at[dst_v.at[j - LAG]],
                              dsem).wait()
      return carry

    lax.fori_loop(0, NCH, chunk, 0)

    def drain(j, carry):
      pltpu.make_async_copy(ew_v.at[j], deg_sp.at[dst_v.at[j]], dsem).wait()
      return carry

    lax.fori_loop(NCH - LAG, NCH, drain, 0)
    plsc.subcore_barrier()
    pltpu.sync_copy(deg_sp.at[pl.ds(s * npart, npart)],
                    deg_hbm.at[c, pl.ds(s * npart, npart)])

  return pl.kernel(
      body,
      out_type=jax.ShapeDtypeStruct((NC, NP), jnp.float32),
      mesh=_sc_mesh(),
      compiler_params=pltpu.CompilerParams(use_tc_tiling_on_sc=False),
      scratch_types=[
          pltpu.VMEM((NCH, CH), jnp.int32),
          pltpu.VMEM((NCH, CH), jnp.float32),
          pltpu.VMEM((640,), jnp.float32),
          pltpu.VMEM_SHARED((NP,), jnp.float32),
          pltpu.SemaphoreType.DMA,
      ],
  )


def _make_agg_kernel(NP, H, NCH, CH):
  npart = NP // NS
  zrows = 80

  NB = 5
  steps = NCH // NB

  def body(hp_hbm, src_hbm, dst_hbm, ew_hbm, s_hbm,
           src_v, dst_v, ew_v, rows_v, sc_v, zb_v, s_sp, gsem, ssem):
    c = lax.axis_index("c")
    s = lax.axis_index("s")
    w = c * NS + s
    z16 = jnp.zeros((LANES,), jnp.float32)
    for r in range(zrows):
      for f in range(H // LANES):
        zb_v[r, pl.ds(f * LANES, LANES)] = z16
    for k in range(npart // zrows):
      pltpu.sync_copy(zb_v, s_sp.at[pl.ds(s * npart + k * zrows, zrows)])
    plsc.subcore_barrier()
    pltpu.sync_copy(src_hbm.at[w], src_v)
    pltpu.sync_copy(dst_hbm.at[w], dst_v)
    pltpu.sync_copy(ew_hbm.at[w], ew_v)

    for b in range(NB):
      pltpu.async_copy(hp_hbm.at[src_v.at[b]], rows_v.at[b], gsem.at[b])

    def scale(j, b):
      for g in range(CH // LANES):
        ew16 = ew_v[j, pl.ds(g * LANES, LANES)]
        for l in range(LANES):
          e = g * LANES + l
          scl = _lane_splat(ew16, l)
          for f in range(H // LANES):
            sl = pl.ds(f * LANES, LANES)
            sc_v[b, e, sl] = rows_v[b, e, sl] * scl

    def step(t, carry):
      for b in range(NB):
        j = t * NB + b
        pltpu.make_async_copy(hp_hbm.at[src_v.at[j]], rows_v.at[b],
                              gsem.at[b]).wait()

        @pl.when(t > 0)
        def _wait_prev_scatter():
          pltpu.make_async_copy(sc_v.at[b], s_sp.at[dst_v.at[j]],
                                ssem.at[b]).wait()

        scale(j, b)

        @pl.when(t + 1 < steps)
        def _next_gather():
          pltpu.async_copy(hp_hbm.at[src_v.at[j + NB]], rows_v.at[b],
                           gsem.at[b])

        pltpu.async_copy(sc_v.at[b], s_sp.at[dst_v.at[j]], ssem.at[b],
                         add=True)
      return carry

    lax.fori_loop(0, steps, step, 0)
    for b in range(NB):
      pltpu.make_async_copy(sc_v.at[b], s_sp.at[dst_v.at[b]],
                            ssem.at[b]).wait()
    plsc.subcore_barrier()
    for k in range(npart // zrows):
      pltpu.sync_copy(s_sp.at[pl.ds(s * npart + k * zrows, zrows)],
                      s_hbm.at[c, pl.ds(s * npart + k * zrows, zrows)])

  return pl.kernel(
      body,
      out_type=jax.ShapeDtypeStruct((NC, NP, H), jnp.float32),
      mesh=_sc_mesh(),
      compiler_params=pltpu.CompilerParams(use_tc_tiling_on_sc=False),
      scratch_types=[
          pltpu.VMEM((NCH, CH), jnp.int32),
          pltpu.VMEM((NCH, CH), jnp.int32),
          pltpu.VMEM((NCH, CH), jnp.float32),
          pltpu.VMEM((NB, CH, H), jnp.float32),
          pltpu.VMEM((NB, CH, H), jnp.float32),
          pltpu.VMEM((zrows, H), jnp.float32),
          pltpu.VMEM_SHARED((NP, H), jnp.float32),
          pltpu.SemaphoreType.DMA((NB,)),
          pltpu.SemaphoreType.DMA((NB,)),
      ],
  )


def _tc_dis_body(degp_ref, dis_ref):
  deg = degp_ref[0] + degp_ref[1] + 1.0
  dis_ref[...] = lax.rsqrt(deg)


def _tc_first_body(x_ref, w_ref, dis_ref, hp_ref):
  h = jnp.dot(x_ref[...], w_ref[...], preferred_element_type=jnp.float32)
  hp_ref[...] = dis_ref[...] * h


def _tc_mid_body(sp_ref, hp_ref, dis_ref, b_ref, w_ref, out_ref):
  H = w_ref.shape[0]
  dis = dis_ref[...]
  agg = dis * (sp_ref[0, :, :H] + sp_ref[1, :, :H] + hp_ref[:, :H]) + b_ref[...]
  h = jnp.maximum(agg, 0.0)
  out_ref[...] = dis * jnp.dot(h, w_ref[...],
                               preferred_element_type=jnp.float32)


def _make_tc_final_body(G):
  def body(sp_ref, hp_ref, dis_ref, b_ref, brow_ref, bcol_ref,
           wout_ref, bout_ref, out_ref, hidden_ref):
    H = b_ref.shape[1]
    dis = dis_ref[...]
    agg = dis * (sp_ref[0, :, :H] + sp_ref[1, :, :H] + hp_ref[:, :H]) + b_ref[...]
    h3 = jnp.maximum(agg, 0.0)
    NP, H = h3.shape
    gids = lax.broadcasted_iota(jnp.int32, (G, NP), 0)
    p = (gids == brow_ref[...]).astype(jnp.float32)
    gsum = jnp.dot(p, h3, preferred_element_type=jnp.float32)
    cnt = jnp.dot(p, jnp.ones((NP, 1), jnp.float32),
                  preferred_element_type=jnp.float32)
    bcol = bcol_ref[...]

    grow = lax.broadcasted_iota(jnp.int32, (G, 1), 0)

    CHR = 256
    acc = jnp.full((G, H), -jnp.inf, jnp.float32)
    for i in range(NP // CHR):
      hc = h3[i * CHR:(i + 1) * CHR]
      bc = bcol[i * CHR:(i + 1) * CHR]
      blo = bc[0, 0]
      bhi = bc[CHR - 1, 0]

      def cbody(g, a, hc=hc, bc=bc):
        m = jnp.max(jnp.where(bc == g, hc, -jnp.inf), axis=0)
        return jnp.maximum(a, jnp.where(grow == g, m[None, :], -jnp.inf))

      acc = lax.fori_loop(blo, bhi + 1, cbody, acc)
    gmax = acc
    gmean = gsum / jnp.maximum(cnt, 1.0)
    hidden = jnp.concatenate([gmax, gmean], axis=1)
    logit = jnp.dot(hidden, wout_ref[...],
                    preferred_element_type=jnp.float32) + bout_ref[...]
    out_ref[...] = jax.nn.sigmoid(logit)
    hidden_ref[...] = hidden
  return body


def _tc_call(body, out_shape):
  return pl.pallas_call(body, out_shape=out_shape)


@jax.jit
def kernel(x, edge_index, edge_weight, batch_index,
           W0, b0, W1, b1, W2, b2, Wout, bout):
  N, F = x.shape
  E = edge_index.shape[1]
  H = W0.shape[1]
  G = 64
  NP = ((N + 1023) // 1024) * 1024
  EPW = E // NW
  CH = 80
  NCH = EPW // CH
  assert E % NW == 0 and EPW % CH == 0 and NP % (NS * 80) == 0

  f32 = jnp.float32
  src_r = edge_index[0].reshape(NW, NCH, CH)
  dst_r = edge_index[1].reshape(NW, NCH, CH)
  ew_r = edge_weight.reshape(NW, NCH, CH)
  x_p = jnp.zeros((NP, F), f32).at[:N].set(x)
  batch_p = jnp.full((NP,), G, jnp.int32).at[:N].set(batch_index)
  brow = batch_p.reshape(1, NP)
  bcol = batch_p.reshape(NP, 1)

  deg_kernel = _make_deg_kernel(NP, NCH, CH)
  agg_kernel = _make_agg_kernel(NP, H, NCH, CH)

  degp = deg_kernel(dst_r, ew_r)
  dis = _tc_call(_tc_dis_body,
                 jax.ShapeDtypeStruct((NP // 128, 128), f32))(
                     degp.reshape(NC, NP // 128, 128))
  dis_col = dis.reshape(NP, 1)

  hp0 = _tc_call(_tc_first_body, jax.ShapeDtypeStruct((NP, H), f32))(
      x_p, W0, dis_col)

  s0 = agg_kernel(hp0, src_r, dst_r, ew_r)
  hp1 = _tc_call(_tc_mid_body, jax.ShapeDtypeStruct((NP, H), f32))(
      s0, hp0, dis_col, b0.reshape(1, H), W1)
  s1 = agg_kernel(hp1, src_r, dst_r, ew_r)
  hp2 = _tc_call(_tc_mid_body, jax.ShapeDtypeStruct((NP, H), f32))(
      s1, hp1, dis_col, b1.reshape(1, H), W2)
  s2 = agg_kernel(hp2, src_r, dst_r, ew_r)

  out, hidden = _tc_call(
      _make_tc_final_body(G),
      (jax.ShapeDtypeStruct((G, 1), f32),
       jax.ShapeDtypeStruct((G, 2 * H), f32)))(
           s2, hp2, dis_col, b2.reshape(1, H), brow, bcol, Wout,
           bout.reshape(1, 1))
  return (out, hidden)
